# Initial kernel scaffold; baseline (speedup 1.0000x reference)
import jax
import jax.numpy as jnp
from jax.experimental import pallas as pl


def kernel(x, edge_index, edge_weight, W1, b1, Wc0, bc0, Wc1, bc1, W2, b2):
    raise NotImplementedError("write your pallas kernel here")


# speedup vs baseline: 53.5187x; 53.5187x over previous
import functools

import jax
import jax.numpy as jnp
from jax import lax
from jax.experimental import pallas as pl
from jax.experimental.pallas import tpu as pltpu
from jax.experimental.pallas import tpu_sc as plsc

N = 100000
E = 3200000
F_IN = 128
H = 16
C = 18

LB = 128
NT = 32
K = 8
NCH = 98
NBT = K * NCH
EPR = NT * NBT
EP = EPR * LB
NP = 100096
RPT = NP // 16
NP2 = 100352
Z1 = NP2 // 16

_mesh = plsc.VectorSubcoreMesh(core_axis_name="c", subcore_axis_name="s")


@functools.partial(
    pl.kernel,
    out_type=jax.ShapeDtypeStruct((2 * NP2,), jnp.float32),
    mesh=_mesh,
    compiler_params=pltpu.CompilerParams(use_tc_tiling_on_sc=False),
    scratch_types=[
        pltpu.VMEM((K, LB), jnp.int32),
        pltpu.VMEM((LB,), jnp.float32),
        pltpu.VMEM((Z1,), jnp.float32),
        pltpu.VMEM_SHARED((NP2,), jnp.float32),
        pltpu.SemaphoreType.DMA,
    ],
)
def _deg_sc(dstp, zeros1, ones1, out, didx, onev, zbuf, acc, sem):
    c = lax.axis_index("c")
    s = lax.axis_index("s")
    wid = s * 2 + c
    pltpu.sync_copy(ones1, onev)
    pltpu.sync_copy(zeros1, zbuf)
    pltpu.sync_copy(zbuf, acc.at[pl.ds(s * Z1, Z1)])
    plsc.subcore_barrier()

    def chunk(ci, _):
        rb = wid * NBT + ci * K
        pltpu.sync_copy(dstp.at[pl.ds(rb, K)], didx)
        descs = []
        for j in range(K):
            descs.append(
                pltpu.async_copy(onev, acc.at[didx.at[j]], sem, add=True))
        for d in descs:
            d.wait()
        return 0

    lax.fori_loop(0, NCH, chunk, 0)
    plsc.subcore_barrier()
    pltpu.sync_copy(acc.at[pl.ds(s * Z1, Z1)], zbuf)
    pltpu.sync_copy(zbuf, out.at[pl.ds(c * NP2 + s * Z1, Z1)])


@functools.partial(
    pl.kernel,
    out_type=jax.ShapeDtypeStruct((2 * NP, H), jnp.float32),
    mesh=_mesh,
    compiler_params=pltpu.CompilerParams(use_tc_tiling_on_sc=False),
    scratch_types=[
        pltpu.VMEM((K, LB), jnp.int32),
        pltpu.VMEM((K, LB), jnp.int32),
        pltpu.VMEM((K, LB, H), jnp.float32),
        pltpu.VMEM_SHARED((NP, H), jnp.float32),
        pltpu.SemaphoreType.DMA,
        pltpu.SemaphoreType.DMA,
    ],
)
def _agg_sc(gtab, srcp, dstp, zeros2, out, sidx, didx, rows, acc, sg, ss):
    c = lax.axis_index("c")
    s = lax.axis_index("s")
    wid = s * 2 + c
    pltpu.sync_copy(zeros2, acc.at[pl.ds(s * RPT, RPT)])
    plsc.subcore_barrier()

    def chunk(ci, _):
        rb = wid * NBT + ci * K
        pltpu.sync_copy(srcp.at[pl.ds(rb, K)], sidx)
        pltpu.sync_copy(dstp.at[pl.ds(rb, K)], didx)
        gd = []
        for j in range(K):
            gd.append(pltpu.async_copy(gtab.at[sidx.at[j]], rows.at[j], sg))
        for d in gd:
            d.wait()
        sd = []
        for j in range(K):
            sd.append(
                pltpu.async_copy(rows.at[j], acc.at[didx.at[j]], ss, add=True))
        for d in sd:
            d.wait()
        return 0

    lax.fori_loop(0, NCH, chunk, 0)
    plsc.subcore_barrier()
    pltpu.sync_copy(acc.at[pl.ds(s * RPT, RPT)],
                    out.at[pl.ds(c * NP + s * RPT, RPT)])


R = 2000


def _mlp_body(x_ref, dinv_ref, w1_ref, b1_ref, wc0_ref, g1_ref):
    h0 = jnp.dot(x_ref[...], w1_ref[...], preferred_element_type=jnp.float32)
    h0 = jnp.maximum(h0 + b1_ref[...], 0.0)
    t1 = jnp.dot(h0, wc0_ref[...], preferred_element_type=jnp.float32)
    g1_ref[...] = t1 * dinv_ref[...]


def _comb_body(p0_ref, p1_ref, g_ref, dinv_ref, b_ref, w_ref, out_ref):
    p = p0_ref[0] + p1_ref[0] + g_ref[...]
    h = jnp.maximum(dinv_ref[...] * p + b_ref[...], 0.0)
    t = jnp.dot(h, w_ref[...], preferred_element_type=jnp.float32)
    out_ref[...] = t * dinv_ref[...]


def _final_body(p0_ref, p1_ref, g_ref, dinv_ref, b_ref, w2_ref, b2_ref,
                out_ref):
    p = p0_ref[0] + p1_ref[0] + g_ref[...]
    h = jnp.maximum(dinv_ref[...] * p + b_ref[...], 0.0)
    o = jnp.dot(h, w2_ref[...], preferred_element_type=jnp.float32)
    o = o + b2_ref[...]
    m = jnp.max(o, axis=-1, keepdims=True)
    lse = jnp.log(jnp.sum(jnp.exp(o - m), axis=-1, keepdims=True)) + m
    out_ref[...] = o - lse


def _row_spec(w):
    return pl.BlockSpec((R, w), lambda i: (i, 0))


def _full_spec(shape):
    return pl.BlockSpec(shape, lambda i: tuple(0 for _ in shape))


def _part_specs():
    return [pl.BlockSpec((1, R, H), lambda i: (0, i, 0)),
            pl.BlockSpec((1, R, H), lambda i: (1, i, 0))]


def kernel(x, edge_index, edge_weight, W1, b1, Wc0, bc0, Wc1, bc1, W2, b2):
    del edge_weight
    src = edge_index[0]
    dst = edge_index[1]
    fill = jnp.full((EP - E,), N, dtype=jnp.int32)
    srcp = jnp.concatenate([src, fill]).reshape(EPR, LB)
    dstp = jnp.concatenate([dst, fill]).reshape(EPR, LB)

    zeros1 = jnp.zeros((Z1,), jnp.float32)
    ones1 = jnp.ones((LB,), jnp.float32)
    zeros2 = jnp.zeros((RPT, H), jnp.float32)
    zrows = jnp.zeros((NP - N, H), jnp.float32)

    degp = _deg_sc(dstp, zeros1, ones1)
    deg = degp[:N] + degp[NP2:NP2 + N] + 1.0
    dinv = lax.rsqrt(deg).reshape(N, 1)

    g1 = pl.pallas_call(
        _mlp_body,
        grid=(N // R,),
        in_specs=[_row_spec(F_IN), _row_spec(1), _full_spec((F_IN, H)),
                  _full_spec((1, H)), _full_spec((H, H))],
        out_specs=_row_spec(H),
        out_shape=jax.ShapeDtypeStruct((N, H), jnp.float32),
    )(x, dinv, W1, b1.reshape(1, H), Wc0)

    gt1 = jnp.concatenate([g1, zrows])
    agg1 = _agg_sc(gt1, srcp, dstp, zeros2).reshape(2, NP, H)

    g2 = pl.pallas_call(
        _comb_body,
        grid=(N // R,),
        in_specs=_part_specs() + [_row_spec(H), _row_spec(1),
                                  _full_spec((1, H)), _full_spec((H, H))],
        out_specs=_row_spec(H),
        out_shape=jax.ShapeDtypeStruct((N, H), jnp.float32),
    )(agg1, agg1, g1, dinv, bc0.reshape(1, H), Wc1)

    gt2 = jnp.concatenate([g2, zrows])
    agg2 = _agg_sc(gt2, srcp, dstp, zeros2).reshape(2, NP, H)

    out = pl.pallas_call(
        _final_body,
        grid=(N // R,),
        in_specs=_part_specs() + [_row_spec(H), _row_spec(1),
                                  _full_spec((1, H)), _full_spec((H, C)),
                                  _full_spec((1, C))],
        out_specs=_row_spec(C),
        out_shape=jax.ShapeDtypeStruct((N, C), jnp.float32),
    )(agg2, agg2, g2, dinv, bc1.reshape(1, H), W2, b2.reshape(1, C))
    return out


# speedup vs baseline: 61.6999x; 1.1529x over previous
import functools

import jax
import jax.numpy as jnp
from jax import lax
from jax.experimental import pallas as pl
from jax.experimental.pallas import tpu as pltpu
from jax.experimental.pallas import tpu_sc as plsc

N = 100000
E = 3200000
F_IN = 128
H = 16
C = 18

LB = 128
NT = 32
K = 4
NCH = 196
NBT = K * NCH
EPR = NT * NBT
EP = EPR * LB
NP = 100096
RPT = NP // 16
NP2 = 100352
Z1 = NP2 // 16

_mesh = plsc.VectorSubcoreMesh(core_axis_name="c", subcore_axis_name="s")


@functools.partial(
    pl.kernel,
    out_type=jax.ShapeDtypeStruct((2 * NP2,), jnp.float32),
    mesh=_mesh,
    compiler_params=pltpu.CompilerParams(use_tc_tiling_on_sc=False),
    scratch_types=[
        pltpu.VMEM((K, LB), jnp.int32),
        pltpu.VMEM((LB,), jnp.float32),
        pltpu.VMEM((Z1,), jnp.float32),
        pltpu.VMEM_SHARED((NP2,), jnp.float32),
        pltpu.SemaphoreType.DMA,
    ],
)
def _deg_sc(dstp, zeros1, ones1, out, didx, onev, zbuf, acc, sem):
    c = lax.axis_index("c")
    s = lax.axis_index("s")
    wid = s * 2 + c
    pltpu.sync_copy(ones1, onev)
    pltpu.sync_copy(zeros1, zbuf)
    pltpu.sync_copy(zbuf, acc.at[pl.ds(s * Z1, Z1)])
    plsc.subcore_barrier()

    def chunk(ci, _):
        rb = wid * NBT + ci * K
        pltpu.sync_copy(dstp.at[pl.ds(rb, K)], didx)
        descs = []
        for j in range(K):
            descs.append(
                pltpu.async_copy(onev, acc.at[didx.at[j]], sem, add=True))
        for d in descs:
            d.wait()
        return 0

    lax.fori_loop(0, NCH, chunk, 0)
    plsc.subcore_barrier()
    pltpu.sync_copy(acc.at[pl.ds(s * Z1, Z1)], zbuf)
    pltpu.sync_copy(zbuf, out.at[pl.ds(c * NP2 + s * Z1, Z1)])


@functools.partial(
    pl.kernel,
    out_type=jax.ShapeDtypeStruct((2 * NP, H), jnp.float32),
    mesh=_mesh,
    compiler_params=pltpu.CompilerParams(use_tc_tiling_on_sc=False),
    scratch_types=[
        pltpu.VMEM((4, K, LB), jnp.int32),
        pltpu.VMEM((4, K, LB), jnp.int32),
        pltpu.VMEM((2, K, LB, H), jnp.float32),
        pltpu.VMEM_SHARED((NP, H), jnp.float32),
        pltpu.SemaphoreType.DMA((2,)),
        pltpu.SemaphoreType.DMA((2,)),
        pltpu.SemaphoreType.DMA((4,)),
    ],
)
def _agg_sc(gtab, srcp, dstp, zeros2, out, sidx, didx, rows, acc, sg, ss, si):
    c = lax.axis_index("c")
    s = lax.axis_index("s")
    wid = s * 2 + c
    base = wid * NBT
    pltpu.sync_copy(zeros2, acc.at[pl.ds(s * RPT, RPT)])
    plsc.subcore_barrier()

    def fire_idx(q, ci):
        rb = base + ci * K
        pltpu.async_copy(srcp.at[pl.ds(rb, K)], sidx.at[q], si.at[q])
        pltpu.async_copy(dstp.at[pl.ds(rb, K)], didx.at[q], si.at[q])

    def wait_idx(q):
        pltpu.make_async_copy(srcp.at[pl.ds(0, K)], sidx.at[q],
                              si.at[q]).wait()
        pltpu.make_async_copy(dstp.at[pl.ds(0, K)], didx.at[q],
                              si.at[q]).wait()

    def fire_gathers(p, q):
        for j in range(K):
            pltpu.async_copy(gtab.at[sidx.at[q, j]], rows.at[p, j], sg.at[p])

    def wait_gathers(p):
        for j in range(K):
            pltpu.make_async_copy(gtab.at[sidx.at[0, j]], rows.at[p, j],
                                  sg.at[p]).wait()

    def fire_scatters(p, q):
        for j in range(K):
            pltpu.async_copy(rows.at[p, j], acc.at[didx.at[q, j]], ss.at[p],
                             add=True)

    def wait_scatters(p):
        for j in range(K):
            pltpu.make_async_copy(rows.at[p, j], acc.at[didx.at[0, j]],
                                  ss.at[p]).wait()

    def step(p, q, prefetch_rb):
        wait_gathers(p)
        wait_scatters(1 - p)
        fire_scatters(p, q)
        wait_idx((q + 1) % 4)
        if prefetch_rb is not None:
            fire_idx((q + 2) % 4, prefetch_rb)
        fire_gathers(1 - p, (q + 1) % 4)

    pltpu.sync_copy(srcp.at[pl.ds(base, K)], sidx.at[0])
    pltpu.sync_copy(dstp.at[pl.ds(base, K)], didx.at[0])
    fire_gathers(0, 0)
    fire_idx(1, 1)
    wait_gathers(0)
    fire_scatters(0, 0)
    wait_idx(1)
    fire_idx(2, 2)
    fire_gathers(1, 1)
    step(1, 1, 3)
    step(0, 2, 4)
    step(1, 3, 5)

    def body(ci4, _):
        ci = 4 * ci4
        step(0, 0, ci + 2)
        step(1, 1, ci + 3)
        step(0, 2, ci + 4)
        step(1, 3, ci + 5)
        return 0

    lax.fori_loop(1, NCH // 4 - 1, body, 0)
    step(0, 0, NCH - 2)
    step(1, 1, NCH - 1)
    step(0, 2, None)
    wait_gathers(1)
    wait_scatters(0)
    fire_scatters(1, 3)
    wait_scatters(1)

    plsc.subcore_barrier()
    pltpu.sync_copy(acc.at[pl.ds(s * RPT, RPT)],
                    out.at[pl.ds(c * NP + s * RPT, RPT)])


R = 2000


def _mlp_body(x_ref, dinv_ref, w1_ref, b1_ref, wc0_ref, g1_ref):
    h0 = jnp.dot(x_ref[...], w1_ref[...], preferred_element_type=jnp.float32)
    h0 = jnp.maximum(h0 + b1_ref[...], 0.0)
    t1 = jnp.dot(h0, wc0_ref[...], preferred_element_type=jnp.float32)
    g1_ref[...] = t1 * dinv_ref[...]


def _comb_body(p0_ref, p1_ref, g_ref, dinv_ref, b_ref, w_ref, out_ref):
    p = p0_ref[0] + p1_ref[0] + g_ref[...]
    h = jnp.maximum(dinv_ref[...] * p + b_ref[...], 0.0)
    t = jnp.dot(h, w_ref[...], preferred_element_type=jnp.float32)
    out_ref[...] = t * dinv_ref[...]


def _final_body(p0_ref, p1_ref, g_ref, dinv_ref, b_ref, w2_ref, b2_ref,
                out_ref):
    p = p0_ref[0] + p1_ref[0] + g_ref[...]
    h = jnp.maximum(dinv_ref[...] * p + b_ref[...], 0.0)
    o = jnp.dot(h, w2_ref[...], preferred_element_type=jnp.float32)
    o = o + b2_ref[...]
    m = jnp.max(o, axis=-1, keepdims=True)
    lse = jnp.log(jnp.sum(jnp.exp(o - m), axis=-1, keepdims=True)) + m
    out_ref[...] = o - lse


def _row_spec(w):
    return pl.BlockSpec((R, w), lambda i: (i, 0))


def _full_spec(shape):
    return pl.BlockSpec(shape, lambda i: tuple(0 for _ in shape))


def _part_specs():
    return [pl.BlockSpec((1, R, H), lambda i: (0, i, 0)),
            pl.BlockSpec((1, R, H), lambda i: (1, i, 0))]


def kernel(x, edge_index, edge_weight, W1, b1, Wc0, bc0, Wc1, bc1, W2, b2):
    del edge_weight
    src = edge_index[0]
    dst = edge_index[1]
    fill = jnp.full((EP - E,), N, dtype=jnp.int32)
    srcp = jnp.concatenate([src, fill]).reshape(EPR, LB)
    dstp = jnp.concatenate([dst, fill]).reshape(EPR, LB)

    zeros1 = jnp.zeros((Z1,), jnp.float32)
    ones1 = jnp.ones((LB,), jnp.float32)
    zeros2 = jnp.zeros((RPT, H), jnp.float32)

    degp = _deg_sc(dstp, zeros1, ones1)
    deg = degp[:N] + degp[NP2:NP2 + N] + 1.0
    dinv = lax.rsqrt(deg).reshape(N, 1)

    g1 = pl.pallas_call(
        _mlp_body,
        grid=(N // R,),
        in_specs=[_row_spec(F_IN), _row_spec(1), _full_spec((F_IN, H)),
                  _full_spec((1, H)), _full_spec((H, H))],
        out_specs=_row_spec(H),
        out_shape=jax.ShapeDtypeStruct((NP, H), jnp.float32),
    )(x, dinv, W1, b1.reshape(1, H), Wc0)
    agg1 = _agg_sc(g1, srcp, dstp, zeros2).reshape(2, NP, H)

    g2 = pl.pallas_call(
        _comb_body,
        grid=(N // R,),
        in_specs=_part_specs() + [_row_spec(H), _row_spec(1),
                                  _full_spec((1, H)), _full_spec((H, H))],
        out_specs=_row_spec(H),
        out_shape=jax.ShapeDtypeStruct((NP, H), jnp.float32),
    )(agg1, agg1, g1, dinv, bc0.reshape(1, H), Wc1)

    agg2 = _agg_sc(g2, srcp, dstp, zeros2).reshape(2, NP, H)

    out = pl.pallas_call(
        _final_body,
        grid=(N // R,),
        in_specs=_part_specs() + [_row_spec(H), _row_spec(1),
                                  _full_spec((1, H)), _full_spec((H, C)),
                                  _full_spec((1, C))],
        out_specs=_row_spec(C),
        out_shape=jax.ShapeDtypeStruct((N, C), jnp.float32),
    )(agg2, agg2, g2, dinv, bc1.reshape(1, H), W2, b2.reshape(1, C))
    return out
